# Initial kernel scaffold; baseline (speedup 1.0000x reference)
import jax
import jax.numpy as jnp
from jax.experimental import pallas as pl


def kernel(query, pos, neg, table):
    raise NotImplementedError("write your pallas kernel here")


# speedup vs baseline: 1.4283x; 1.4283x over previous
import functools

import jax
import jax.numpy as jnp
from jax import lax
from jax.experimental import pallas as pl
from jax.experimental.pallas import tpu as pltpu
from jax.experimental.pallas import tpu_sc as plsc

D = 64
B = 16384
SEQ = 20
NC = 2
NS = 16
NW = NC * NS
G = 32
CHUNKS = B // (NW * G)
STAGES = CHUNKS * 3
ROWS = G * SEQ
NIDX = ROWS // 128
NIDXP = 8
INV400 = 1.0 / (SEQ * SEQ)


@functools.partial(
    pl.kernel,
    out_type=jax.ShapeDtypeStruct((B,), jnp.float32),
    mesh=plsc.VectorSubcoreMesh(core_axis_name="c", subcore_axis_name="s"),
    compiler_params=pltpu.CompilerParams(
        needs_layout_passes=False, use_tc_tiling_on_sc=False),
    scratch_types=[
        pltpu.VMEM((NIDXP, 128), jnp.int32),
        pltpu.VMEM((ROWS, D), jnp.float32),
        pltpu.VMEM((3, G, D), jnp.float32),
        pltpu.VMEM((CHUNKS * G,), jnp.float32),
        pltpu.SemaphoreType.DMA,
    ],
)
def _feel_sc(table_hbm, idx_hbm, out_hbm, idx_v, rows_v, acc_v, out_v, sem):
    wid = lax.axis_index("s") * NC + lax.axis_index("c")
    base = wid * (STAGES * NIDXP)

    def stage(t, carry):
        k = lax.rem(t, 3)
        c = lax.div(t, 3)
        pltpu.sync_copy(idx_hbm.at[pl.ds(base + t * NIDXP, NIDXP)], idx_v)
        cps = [
            pltpu.async_copy(
                table_hbm.at[idx_v.at[j]],
                rows_v.at[pl.ds(j * 128, 128)],
                sem,
            )
            for j in range(NIDX)
        ]
        for cp in cps:
            cp.wait()

        def acc_body(e, carry2):
            for blk in range(D // 16):
                sl = pl.ds(blk * 16, 16)
                s = rows_v[e * SEQ, sl]
                for q in range(1, SEQ):
                    s = s + rows_v[e * SEQ + q, sl]
                acc_v[k, e, sl] = s
            return carry2

        lax.fori_loop(0, G, acc_body, 0)

        @pl.when(k == 2)
        def _dots():
            lane = lax.iota(jnp.int32, 16)
            tbl = [jnp.full((16,), i, jnp.int32) for i in range(3)]
            for h in range(G // 16):
                e_idx = lane + (h * 16)
                d = jnp.zeros((16,), jnp.float32)
                for dim in range(D):
                    dimv = jnp.full((16,), dim, jnp.int32)
                    qv = plsc.load_gather(acc_v, [tbl[0], e_idx, dimv])
                    pv = plsc.load_gather(acc_v, [tbl[1], e_idx, dimv])
                    nv = plsc.load_gather(acc_v, [tbl[2], e_idx, dimv])
                    d = d + qv * (pv - nv)
                out_v[pl.ds(c * G + h * 16, 16)] = jnp.maximum(
                    0.0, 1.0 - d * INV400)

        return carry

    lax.fori_loop(0, STAGES, stage, 0)
    pltpu.sync_copy(out_v, out_hbm.at[pl.ds(wid * (CHUNKS * G), CHUNKS * G)])


def kernel(query, pos, neg, table):
    idx = jnp.stack([
        query.astype(jnp.int32),
        pos.astype(jnp.int32),
        neg.astype(jnp.int32),
    ])
    idx = idx.reshape(3, NW, CHUNKS, G * SEQ)
    idx = idx.transpose(1, 2, 0, 3).reshape(NW * STAGES, NIDX, 128)
    idx = jnp.pad(idx, ((0, 0), (0, NIDXP - NIDX), (0, 0)))
    idx = idx.reshape(NW * STAGES * NIDXP, 128)
    return _feel_sc(table, idx)


# speedup vs baseline: 1.6342x; 1.1441x over previous
import functools

import jax
import jax.numpy as jnp
from jax import lax
from jax.experimental import pallas as pl
from jax.experimental.pallas import tpu as pltpu
from jax.experimental.pallas import tpu_sc as plsc

D = 64
B = 16384
SEQ = 20
NC = 2
NS = 16
NW = NC * NS
G = 32
CHUNKS = B // (NW * G)
ROWS = G * SEQ
NIDX = ROWS // 128
IDXROWS = B * SEQ // 128
INV400 = 1.0 / (SEQ * SEQ)
NSTG = 6


@functools.partial(
    pl.kernel,
    out_type=jax.ShapeDtypeStruct((B,), jnp.float32),
    mesh=plsc.VectorSubcoreMesh(core_axis_name="c", subcore_axis_name="s"),
    compiler_params=pltpu.CompilerParams(
        needs_layout_passes=False, use_tc_tiling_on_sc=False),
    scratch_types=[
        pltpu.VMEM((2, NIDX, 128), jnp.int32),
        pltpu.VMEM((2, ROWS, D), jnp.float32),
        pltpu.VMEM((3, G, D), jnp.float32),
        pltpu.VMEM((CHUNKS * G,), jnp.float32),
        pltpu.SemaphoreType.DMA,
        pltpu.SemaphoreType.DMA,
    ],
)
def _feel_sc(table_hbm, q_hbm, p_hbm, n_hbm, out_hbm,
             idx_v, rows_v, acc_v, out_v, sem_g, sem_i):
    wid = lax.axis_index("s") * NC + lax.axis_index("c")
    idx_base = wid * (CHUNKS * NIDX)
    idx_refs = [q_hbm, p_hbm, n_hbm]

    def idx_src(s, cc):
        chunk = cc * 2 + (0 if s < 3 else 1)
        return idx_refs[s % 3].at[pl.ds(idx_base + chunk * NIDX, NIDX)]

    def fire_idx(s, cc):
        return pltpu.async_copy(idx_src(s % NSTG, cc), idx_v.at[(s % NSTG) % 2],
                                sem_i)

    def fire_gathers(s, cc):
        par = s % 2
        return [
            pltpu.async_copy(
                table_hbm.at[idx_v.at[par, j]],
                rows_v.at[par, pl.ds(j * 128, 128)],
                sem_g,
            )
            for j in range(NIDX)
        ]

    fire_idx(0, 0).wait()
    fire_gathers(0, 0)
    fire_idx(1, 0)

    def dchunk(cc, carry):
        for s in range(NSTG):
            k = s % 3
            chunk = cc * 2 + (0 if s < 3 else 1)
            par = s % 2

            for j in range(NIDX):
                pltpu.make_async_copy(
                    table_hbm.at[idx_v.at[par, j]],
                    rows_v.at[par, pl.ds(j * 128, 128)],
                    sem_g,
                ).wait()

            def _issue_next():
                nxt = s + 1
                ncc = cc + (1 if nxt >= NSTG else 0)
                pltpu.make_async_copy(idx_src(nxt % NSTG, ncc),
                                      idx_v.at[(nxt % NSTG) % 2],
                                      sem_i).wait()
                fire_gathers(nxt, ncc)

            def _issue_idx2():
                n2 = s + 2
                ncc2 = cc + (1 if n2 >= NSTG else 0)
                fire_idx(n2, ncc2)

            if s == NSTG - 1:
                pl.when(cc < CHUNKS // 2 - 1)(_issue_next)
            else:
                _issue_next()
            if s >= NSTG - 2:
                pl.when(cc < CHUNKS // 2 - 1)(_issue_idx2)
            else:
                _issue_idx2()

            def acc_body(e, carry2):
                for blk in range(D // 16):
                    sl = pl.ds(blk * 16, 16)
                    v = rows_v[par, e * SEQ, sl]
                    for q in range(1, SEQ):
                        v = v + rows_v[par, e * SEQ + q, sl]
                    acc_v[k, e, sl] = v
                return carry2

            lax.fori_loop(0, G, acc_body, 0)

            if k == 2:
                lane = lax.iota(jnp.int32, 16)
                tbl = [jnp.full((16,), i, jnp.int32) for i in range(3)]
                for h in range(G // 16):
                    e_idx = lane + (h * 16)
                    d = jnp.zeros((16,), jnp.float32)
                    for dim in range(D):
                        dimv = jnp.full((16,), dim, jnp.int32)
                        qv = plsc.load_gather(acc_v, [tbl[0], e_idx, dimv])
                        pv = plsc.load_gather(acc_v, [tbl[1], e_idx, dimv])
                        nv = plsc.load_gather(acc_v, [tbl[2], e_idx, dimv])
                        d = d + qv * (pv - nv)
                    out_v[pl.ds(chunk * G + h * 16, 16)] = jnp.maximum(
                        0.0, 1.0 - d * INV400)
        return carry

    lax.fori_loop(0, CHUNKS // 2, dchunk, 0)
    pltpu.sync_copy(out_v, out_hbm.at[pl.ds(wid * (CHUNKS * G), CHUNKS * G)])


def kernel(query, pos, neg, table):
    q = query.astype(jnp.int32).reshape(IDXROWS, 128)
    p = pos.astype(jnp.int32).reshape(IDXROWS, 128)
    n = neg.astype(jnp.int32).reshape(IDXROWS, 128)
    return _feel_sc(table, q, p, n)
